# Initial kernel scaffold; baseline (speedup 1.0000x reference)
import jax
import jax.numpy as jnp
from jax.experimental import pallas as pl


def kernel(x, adj, mask, n_nodes, W, b, pooling):
    raise NotImplementedError("write your pallas kernel here")


# speedup vs baseline: 1.1979x; 1.1979x over previous
import functools

import jax
import jax.numpy as jnp
from jax import lax
from jax.experimental import pallas as pl
from jax.experimental.pallas import tpu as pltpu
from jax.experimental.pallas import tpu_sc as plsc

N = 10000
F = 128
R1 = 400
R2 = 200
CH = N // 16

_I32_MIN = jnp.int32(-2147483648)


def _conv_body(x_ref, adj_ref, w_ref, b_ref, p_ref, z_ref, y_ref, h_ref):
    @pl.when(pl.program_id(0) == 0)
    def _():
        h_ref[...] = jnp.dot(x_ref[...], w_ref[...],
                             preferred_element_type=jnp.float32)

    zb = jnp.dot(adj_ref[...], h_ref[...],
                 preferred_element_type=jnp.float32) + b_ref[...]
    z_ref[...] = zb
    p = p_ref[...]
    pn = jnp.sqrt(jnp.sum(p * p))
    y_ref[...] = jnp.dot(zb, p, preferred_element_type=jnp.float32) / pn


def _conv_call(x2, adj2, W, b2, p):
    return pl.pallas_call(
        _conv_body,
        grid=(N // R1,),
        in_specs=[
            pl.BlockSpec((N, F), lambda i: (0, 0)),
            pl.BlockSpec((R1, N), lambda i: (i, 0)),
            pl.BlockSpec((F, F), lambda i: (0, 0)),
            pl.BlockSpec((1, F), lambda i: (0, 0)),
            pl.BlockSpec((F, 1), lambda i: (0, 0)),
        ],
        out_specs=[
            pl.BlockSpec((R1, F), lambda i: (i, 0)),
            pl.BlockSpec((R1, 1), lambda i: (i, 0)),
        ],
        out_shape=[
            jax.ShapeDtypeStruct((N, F), jnp.float32),
            jax.ShapeDtypeStruct((N, 1), jnp.float32),
        ],
        scratch_shapes=[pltpu.VMEM((N, F), jnp.float32)],
    )(x2, adj2, W, b2, p)


def _sel_body(y_hbm, m_hbm, k_hbm, out_hbm, y_v, m_v, w_v, o_v, hist_v, k_v,
              tmp_v):
    @pl.when((lax.axis_index("c") == 0) & (lax.axis_index("s") == 0))
    def _():
        pltpu.sync_copy(y_hbm, y_v)
        pltpu.sync_copy(m_hbm, m_v)
        pltpu.sync_copy(k_hbm, k_v)
        lane = lax.iota(jnp.int32, 16)
        ones16 = jnp.ones((16,), jnp.int32)
        zeros16 = jnp.zeros((16,), jnp.int32)

        def csum16(x):
            c = x
            for s in (1, 2, 4, 8):
                tmp_v[...] = c
                g = plsc.load_gather(tmp_v, [jnp.maximum(lane - s, 0)])
                c = c + jnp.where(lane >= s, g, 0)
            return c

        def splat_last(c):
            tmp_v[...] = c
            return plsc.load_gather(tmp_v, [zeros16 + 15])

        def build(i, c):
            sl = pl.ds(i * 16, 16)
            yv = y_v[sl]
            yv = jnp.where(yv == 0.0, jnp.float32(0.0), yv)
            bb = lax.bitcast_convert_type(yv, jnp.int32)
            u = jnp.where(bb < 0, ~bb, bb ^ _I32_MIN)
            u = jnp.where(m_v[sl] == 1.0, u, jnp.int32(-1))
            w_v[sl] = u
            return c

        lax.fori_loop(0, CH, build, jnp.int32(0))

        prefix = zeros16
        kk = k_v[...]
        for r in range(8):
            shift = 28 - 4 * r
            hm = (0xFFFFFFFF << (shift + 4)) & 0xFFFFFFFF
            himask = jnp.int32(hm - (1 << 32) if hm >= (1 << 31) else hm)
            for j in range(16):
                hist_v[pl.ds(j * 16, 16)] = zeros16

            def cnt(i, c, himask=himask, prefix=prefix, shift=shift):
                sl = pl.ds(i * 16, 16)
                v = w_v[sl]
                match = (v & himask) == prefix
                bucket = lax.shift_right_logical(v, shift) & 15
                plsc.addupdate_scatter(hist_v, [lane * 16 + bucket], ones16,
                                       mask=match)
                return c

            lax.fori_loop(0, CH, cnt, jnp.int32(0))
            c16 = hist_v[pl.ds(0, 16)]
            for j in range(1, 16):
                c16 = c16 + hist_v[pl.ds(j * 16, 16)]
            cum = csum16(c16)
            d = splat_last(csum16((cum < kk).astype(jnp.int32)))
            tmp_v[...] = cum
            prev = plsc.load_gather(tmp_v, [jnp.maximum(d - 1, 0)])
            prev = jnp.where(d >= 1, prev, 0)
            kk = kk - prev
            prefix = prefix | (d << shift)

        t = prefix
        tb = t ^ _I32_MIN

        def fin(i, c):
            sl = pl.ds(i * 16, 16)
            v = w_v[sl]
            eq = v == t
            ei = eq.astype(jnp.int32)
            cs = csum16(ei)
            rank = c + cs
            rem = ((v ^ _I32_MIN) < tb) | (eq & (rank <= kk))
            o_v[sl] = m_v[sl] * jnp.where(rem, 0.0, 1.0).astype(jnp.float32)
            return c + splat_last(cs)

        lax.fori_loop(0, CH, fin, zeros16)
        pltpu.sync_copy(o_v, out_hbm)


def _select_call(y1d, m1d, k16):
    mesh = plsc.VectorSubcoreMesh(core_axis_name="c", subcore_axis_name="s")
    fn = functools.partial(
        pl.kernel,
        mesh=mesh,
        compiler_params=pltpu.CompilerParams(needs_layout_passes=False),
        out_type=jax.ShapeDtypeStruct((N,), jnp.float32),
        scratch_types=[
            pltpu.VMEM((N,), jnp.float32),
            pltpu.VMEM((N,), jnp.float32),
            pltpu.VMEM((N,), jnp.int32),
            pltpu.VMEM((N,), jnp.float32),
            pltpu.VMEM((256,), jnp.int32),
            pltpu.VMEM((16,), jnp.int32),
            pltpu.VMEM((16,), jnp.int32),
        ],
    )(_sel_body)
    return fn(y1d, m1d, k16)


def _fin_body(adj_ref, z_ref, y_ref, mr_ref, mc_ref, adj_out_ref, x_out_ref):
    mr = mr_ref[...]
    adj_out_ref[...] = adj_ref[...] * mr * mc_ref[...]
    x_out_ref[...] = z_ref[...] * jnp.tanh(y_ref[...]) * mr


def _finalize_call(adj2, z, y, mr, mc):
    return pl.pallas_call(
        _fin_body,
        grid=(N // R2,),
        in_specs=[
            pl.BlockSpec((R2, N), lambda i: (i, 0)),
            pl.BlockSpec((R2, F), lambda i: (i, 0)),
            pl.BlockSpec((R2, 1), lambda i: (i, 0)),
            pl.BlockSpec((R2, 1), lambda i: (i, 0)),
            pl.BlockSpec((1, N), lambda i: (0, 0)),
        ],
        out_specs=[
            pl.BlockSpec((R2, N), lambda i: (i, 0)),
            pl.BlockSpec((R2, F), lambda i: (i, 0)),
        ],
        out_shape=[
            jax.ShapeDtypeStruct((N, N), jnp.float32),
            jax.ShapeDtypeStruct((N, F), jnp.float32),
        ],
    )(adj2, z, y, mr, mc)


def kernel(x, adj, mask, n_nodes, W, b, pooling):
    x2 = x[0]
    adj2 = adj[0]
    m1 = mask[0]
    n_remove = (n_nodes.astype(jnp.float32) * 0.5).astype(jnp.int32)
    n_new = n_nodes - n_remove

    z, y = _conv_call(x2, adj2, W, b.reshape(1, F), pooling)
    k16 = jnp.broadcast_to(n_remove, (16,)).astype(jnp.int32)
    mask_new = _select_call(y.reshape(N), m1, k16)
    adj_out, x_out = _finalize_call(adj2, z, y, mask_new.reshape(N, 1),
                                    mask_new.reshape(1, N))
    return (x_out[None], adj_out[None], mask_new.reshape(1, N), n_new)


# speedup vs baseline: 1.2439x; 1.0384x over previous
import functools

import jax
import jax.numpy as jnp
from jax import lax
from jax.experimental import pallas as pl
from jax.experimental.pallas import tpu as pltpu
from jax.experimental.pallas import tpu_sc as plsc

N = 10000
F = 128
R1 = 400
R2 = 200
CH = N // 16

_I32_MIN = jnp.int32(-2147483648)


def _conv_body(x_ref, adj_ref, w_ref, b_ref, p_ref, z_ref, y_ref, h_ref):
    @pl.when(pl.program_id(0) == 0)
    def _():
        h_ref[...] = jnp.dot(x_ref[...], w_ref[...],
                             preferred_element_type=jnp.float32)

    zb = jnp.dot(adj_ref[...], h_ref[...],
                 preferred_element_type=jnp.float32) + b_ref[...]
    z_ref[...] = zb
    p = p_ref[...]
    pn = jnp.sqrt(jnp.sum(p * p))
    y_ref[...] = jnp.dot(zb, p, preferred_element_type=jnp.float32) / pn


def _conv_call(x2, adj2, W, b2, p):
    return pl.pallas_call(
        _conv_body,
        grid=(N // R1,),
        in_specs=[
            pl.BlockSpec((N, F), lambda i: (0, 0)),
            pl.BlockSpec((R1, N), lambda i: (i, 0)),
            pl.BlockSpec((F, F), lambda i: (0, 0)),
            pl.BlockSpec((1, F), lambda i: (0, 0)),
            pl.BlockSpec((F, 1), lambda i: (0, 0)),
        ],
        out_specs=[
            pl.BlockSpec((R1, F), lambda i: (i, 0)),
            pl.BlockSpec((R1, 1), lambda i: (i, 0)),
        ],
        out_shape=[
            jax.ShapeDtypeStruct((N, F), jnp.float32),
            jax.ShapeDtypeStruct((N, 1), jnp.float32),
        ],
        scratch_shapes=[pltpu.VMEM((N, F), jnp.float32)],
    )(x2, adj2, W, b2, p)


def _sel_body(y_hbm, m_hbm, k_hbm, out_hbm, y_v, m_v, w_v, o_v, hist_v, k_v,
              tmp_v, cums_v, eqi_v):
    @pl.when((lax.axis_index("c") == 0) & (lax.axis_index("s") == 0))
    def _():
        pltpu.sync_copy(y_hbm, y_v)
        pltpu.sync_copy(m_hbm, m_v)
        pltpu.sync_copy(k_hbm, k_v)
        lane = lax.iota(jnp.int32, 16)
        ones16 = jnp.ones((16,), jnp.int32)
        zeros16 = jnp.zeros((16,), jnp.int32)
        k = k_v[pl.ds(0, 16)][0]

        def csum16(x):
            c = x
            for s in (1, 2, 4, 8):
                tmp_v[...] = c
                g = plsc.load_gather(tmp_v, [jnp.maximum(lane - s, 0)])
                c = c + jnp.where(lane >= s, g, 0)
            return c

        for j in range(256):
            hist_v[pl.ds(j * 16, 16)] = zeros16

        def build(i, c):
            sl = pl.ds(i * 16, 16)
            yv = y_v[sl]
            yv = jnp.where(yv == 0.0, jnp.float32(0.0), yv)
            bb = lax.bitcast_convert_type(yv, jnp.int32)
            u = jnp.where(bb < 0, ~bb, bb ^ _I32_MIN)
            u = jnp.where(m_v[sl] == 1.0, u, jnp.int32(-1))
            w_v[sl] = u
            bucket = lax.shift_right_logical(u, 24)
            plsc.addupdate_scatter(hist_v, [lane * 256 + bucket], ones16)
            return c

        lax.fori_loop(0, CH, build, jnp.int32(0))

        def pick_bucket(kk):
            carry = jnp.int32(0)
            d = jnp.int32(0)
            for g in range(16):
                tot = hist_v[pl.ds(g * 16, 16)]
                for l in range(1, 16):
                    tot = tot + hist_v[pl.ds(l * 256 + g * 16, 16)]
                cum = csum16(tot) + carry
                cums_v[pl.ds(g * 16, 16)] = cum
                d = d + plsc.all_reduce_population_count(cum < kk)[0]
                carry = cum[15]
            prev = plsc.load_gather(cums_v, [jnp.maximum(zeros16 + d - 1, 0)])
            prev = jnp.where(d >= 1, prev[0], jnp.int32(0))
            return d, prev

        prefix = jnp.int32(0)
        kk = k
        for r in range(4):
            shift = 24 - 8 * r
            if r > 0:
                for j in range(256):
                    hist_v[pl.ds(j * 16, 16)] = zeros16
                hm = (0xFFFFFFFF << (shift + 8)) & 0xFFFFFFFF
                himask = jnp.int32(hm - (1 << 32) if hm >= (1 << 31) else hm)

                def cnt(i, c, himask=himask, prefix=prefix, shift=shift):
                    sl = pl.ds(i * 16, 16)
                    v = w_v[sl]
                    match = (v & himask) == prefix
                    bucket = lax.shift_right_logical(v, shift) & 255
                    plsc.addupdate_scatter(hist_v, [lane * 256 + bucket],
                                           ones16, mask=match)
                    return c

                lax.fori_loop(0, CH, cnt, jnp.int32(0))
            d, prev = pick_bucket(kk)
            kk = kk - prev
            prefix = prefix | (d << shift)

        t = prefix
        tb = t ^ _I32_MIN

        def fin(i, off):
            sl = pl.ds(i * 16, 16)
            v = w_v[sl]
            eq = v == t
            rem = (v ^ _I32_MIN) < tb
            o_v[sl] = m_v[sl] * jnp.where(rem, 0.0, 1.0).astype(jnp.float32)
            pc = plsc.all_reduce_population_count(eq)[0]

            def compact():
                _, idx = plsc.sort_key_val(
                    jnp.where(eq, 0, 1).astype(jnp.int32), lane + i * 16)
                plsc.store_scatter(eqi_v, [off + lane], idx, mask=lane < pc)

            pl.when(pc > 0)(compact)
            return off + pc

        m3 = lax.fori_loop(0, CH, fin, jnp.int32(0))

        @pl.when(m3 <= 16)
        def _tiny():
            ev = eqi_v[pl.ds(0, 16)]
            ev = jnp.where(lane < m3, ev, jnp.int32(0x7FFFFFFF))
            sv, _ = plsc.sort_key_val(ev, ev)
            plsc.store_scatter(o_v, [sv], jnp.zeros((16,), jnp.float32),
                               mask=lane < kk)

        @pl.when(m3 > 16)
        def _big():
            def fin2(i, c):
                sl = pl.ds(i * 16, 16)
                v = w_v[sl]
                ei = (v == t).astype(jnp.int32)
                cs = csum16(ei)
                rank = c + cs
                rem_eq = (v == t) & (rank <= kk)
                o_v[sl] = jnp.where(rem_eq, 0.0, o_v[sl]).astype(jnp.float32)
                return c + cs[15]

            lax.fori_loop(0, CH, fin2, jnp.int32(0))

        pltpu.sync_copy(o_v, out_hbm)


def _select_call(y1d, m1d, k16):
    mesh = plsc.VectorSubcoreMesh(core_axis_name="c", subcore_axis_name="s")
    fn = functools.partial(
        pl.kernel,
        mesh=mesh,
        compiler_params=pltpu.CompilerParams(needs_layout_passes=False),
        out_type=jax.ShapeDtypeStruct((N,), jnp.float32),
        scratch_types=[
            pltpu.VMEM((N,), jnp.float32),
            pltpu.VMEM((N,), jnp.float32),
            pltpu.VMEM((N,), jnp.int32),
            pltpu.VMEM((N,), jnp.float32),
            pltpu.VMEM((4096,), jnp.int32),
            pltpu.VMEM((16,), jnp.int32),
            pltpu.VMEM((16,), jnp.int32),
            pltpu.VMEM((256,), jnp.int32),
            pltpu.VMEM((N + 16,), jnp.int32),
        ],
    )(_sel_body)
    return fn(y1d, m1d, k16)


def _fin_body(adj_ref, z_ref, y_ref, mr_ref, mc_ref, adj_out_ref, x_out_ref):
    mr = mr_ref[...]
    adj_out_ref[...] = adj_ref[...] * mr * mc_ref[...]
    x_out_ref[...] = z_ref[...] * jnp.tanh(y_ref[...]) * mr


def _finalize_call(adj2, z, y, mr, mc):
    return pl.pallas_call(
        _fin_body,
        grid=(N // R2,),
        in_specs=[
            pl.BlockSpec((R2, N), lambda i: (i, 0)),
            pl.BlockSpec((R2, F), lambda i: (i, 0)),
            pl.BlockSpec((R2, 1), lambda i: (i, 0)),
            pl.BlockSpec((R2, 1), lambda i: (i, 0)),
            pl.BlockSpec((1, N), lambda i: (0, 0)),
        ],
        out_specs=[
            pl.BlockSpec((R2, N), lambda i: (i, 0)),
            pl.BlockSpec((R2, F), lambda i: (i, 0)),
        ],
        out_shape=[
            jax.ShapeDtypeStruct((N, N), jnp.float32),
            jax.ShapeDtypeStruct((N, F), jnp.float32),
        ],
    )(adj2, z, y, mr, mc)


def kernel(x, adj, mask, n_nodes, W, b, pooling):
    x2 = x[0]
    adj2 = adj[0]
    m1 = mask[0]
    n_remove = (n_nodes.astype(jnp.float32) * 0.5).astype(jnp.int32)
    n_new = n_nodes - n_remove

    z, y = _conv_call(x2, adj2, W, b.reshape(1, F), pooling)
    k16 = jnp.broadcast_to(n_remove, (16,)).astype(jnp.int32)
    mask_new = _select_call(y.reshape(N), m1, k16)
    adj_out, x_out = _finalize_call(adj2, z, y, mask_new.reshape(N, 1),
                                    mask_new.reshape(1, N))
    return (x_out[None], adj_out[None], mask_new.reshape(1, N), n_new)
